# Initial kernel scaffold; baseline (speedup 1.0000x reference)
import jax
import jax.numpy as jnp
from jax.experimental import pallas as pl


def kernel(bbox_targets, bbox_weights, labels):
    raise NotImplementedError("write your pallas kernel here")


# speedup vs baseline: 1.5280x; 1.5280x over previous
import functools

import jax
import jax.numpy as jnp
from jax import lax
from jax.experimental import pallas as pl
from jax.experimental.pallas import tpu as pltpu
from jax.experimental.pallas import tpu_sc as plsc

M = 300
NUM_CLASSES = 80
BOX_DIM = 4
OUT_W = NUM_CLASSES * BOX_DIM
ROWS_PER_WORKER = 10
NUM_WORKERS = 32
ACTIVE_WORKERS = M // ROWS_PER_WORKER
LANES = 16
CHUNKS = OUT_W // LANES


def _body(t_hbm, w_hbm, labels_hbm, out_t_hbm, out_w_hbm,
          labels_v, mask_tab, t_v, w_v, out_t_v, out_w_v):
    nc = 2
    wid = lax.axis_index("s") * nc + lax.axis_index("c")
    base = wid * ROWS_PER_WORKER

    iota = lax.broadcasted_iota(jnp.int32, (LANES,), 0)
    iota4 = iota & 3
    zeros = jnp.zeros((LANES,), jnp.float32)
    ones = jnp.ones((LANES,), jnp.float32)

    @pl.when(wid < ACTIVE_WORKERS)
    def _():
        pltpu.sync_copy(labels_hbm, labels_v)
        pltpu.sync_copy(t_hbm.at[pl.ds(base, ROWS_PER_WORKER)], t_v)
        pltpu.sync_copy(w_hbm.at[pl.ds(base, ROWS_PER_WORKER)], w_v)

        for i in range(NUM_CLASSES // LANES):
            mask_tab[pl.ds(i * LANES, LANES)] = zeros
        lab_vec = plsc.load_gather(labels_v, [iota & 7])
        plsc.store_scatter(mask_tab, [lab_vec], ones)

        m = [plsc.load_gather(mask_tab, [(i * LANES + iota) >> 2])
             for i in range(CHUNKS)]

        for r in range(ROWS_PER_WORKER):
            t_row = plsc.load_gather(t_v, [jnp.full((LANES,), r, jnp.int32),
                                           iota4])
            for v in range(CHUNKS):
                out_t_v[r, pl.ds(v * LANES, LANES)] = m[v] * t_row

        for r in range(ROWS_PER_WORKER):
            for v in range(CHUNKS):
                out_w_v[r, pl.ds(v * LANES, LANES)] = zeros
        for r in range(ROWS_PER_WORKER):
            r_g = base + r
            r_eff16 = jnp.minimum(jnp.full((LANES,), r_g, jnp.int32),
                                  NUM_CLASSES - 1)
            w_row = plsc.load_gather(w_v, [jnp.full((LANES,), r, jnp.int32),
                                           iota4])
            mval = plsc.load_gather(mask_tab, [r_eff16])
            col = r_eff16 * BOX_DIM + iota4
            lane_mask = (iota < BOX_DIM) & (jnp.full((LANES,), r_g, jnp.int32)
                                            < NUM_CLASSES)
            plsc.store_scatter(out_w_v,
                               [jnp.full((LANES,), r, jnp.int32), col],
                               w_row * mval, mask=lane_mask)

        pltpu.sync_copy(out_t_v, out_t_hbm.at[pl.ds(base, ROWS_PER_WORKER)])
        pltpu.sync_copy(out_w_v, out_w_hbm.at[pl.ds(base, ROWS_PER_WORKER)])


@jax.jit
def kernel(bbox_targets, bbox_weights, labels):
    mesh = plsc.VectorSubcoreMesh(core_axis_name="c", subcore_axis_name="s")
    return pl.kernel(
        _body,
        out_type=(jax.ShapeDtypeStruct((M, OUT_W), jnp.float32),
                  jax.ShapeDtypeStruct((M, OUT_W), jnp.float32)),
        mesh=mesh,
        compiler_params=pltpu.CompilerParams(use_tc_tiling_on_sc=False,
                                             needs_layout_passes=False),
        scratch_types=[
            pltpu.VMEM((8,), jnp.int32),
            pltpu.VMEM((NUM_CLASSES,), jnp.float32),
            pltpu.VMEM((ROWS_PER_WORKER, BOX_DIM), jnp.float32),
            pltpu.VMEM((ROWS_PER_WORKER, BOX_DIM), jnp.float32),
            pltpu.VMEM((ROWS_PER_WORKER, OUT_W), jnp.float32),
            pltpu.VMEM((ROWS_PER_WORKER, OUT_W), jnp.float32),
        ],
    )(bbox_targets, bbox_weights, labels)


# speedup vs baseline: 1.5860x; 1.0380x over previous
import functools

import jax
import jax.numpy as jnp
from jax import lax
from jax.experimental import pallas as pl
from jax.experimental.pallas import tpu as pltpu
from jax.experimental.pallas import tpu_sc as plsc

M = 300
NUM_CLASSES = 80
BOX_DIM = 4
OUT_W = NUM_CLASSES * BOX_DIM
ROWS_PER_WORKER = 10
NUM_WORKERS = 32
ACTIVE_WORKERS = M // ROWS_PER_WORKER
LANES = 16
CHUNKS = OUT_W // LANES


def _body(t_hbm, w_hbm, labels_hbm, out_t_hbm, out_w_hbm,
          labels_v, mask_tab, t_v, w_v, out_t_v, out_w_v,
          sem_lab, sem_t, sem_w, sem_out):
    nc = 2
    wid = lax.axis_index("s") * nc + lax.axis_index("c")
    base = wid * ROWS_PER_WORKER

    iota = lax.broadcasted_iota(jnp.int32, (LANES,), 0)
    iota4 = iota & 3
    zeros = jnp.zeros((LANES,), jnp.float32)
    ones = jnp.ones((LANES,), jnp.float32)

    @pl.when(wid < ACTIVE_WORKERS)
    def _():
        lab_cp = pltpu.async_copy(labels_hbm, labels_v, sem_lab)
        t_cp = pltpu.async_copy(t_hbm.at[pl.ds(base, ROWS_PER_WORKER)],
                                t_v, sem_t)
        w_cp = pltpu.async_copy(w_hbm.at[pl.ds(base, ROWS_PER_WORKER)],
                                w_v, sem_w)

        for i in range(NUM_CLASSES // LANES):
            mask_tab[pl.ds(i * LANES, LANES)] = zeros
        lab_cp.wait()
        lab_vec = plsc.load_gather(labels_v, [iota & 7])
        plsc.store_scatter(mask_tab, [lab_vec], ones)

        m = [plsc.load_gather(mask_tab, [(i * LANES + iota) >> 2])
             for i in range(CHUNKS)]

        t_cp.wait()
        for r in range(ROWS_PER_WORKER):
            t_row = plsc.load_gather(t_v, [jnp.full((LANES,), r, jnp.int32),
                                           iota4])
            for v in range(CHUNKS):
                out_t_v[r, pl.ds(v * LANES, LANES)] = m[v] * t_row

        for r in range(ROWS_PER_WORKER):
            for v in range(CHUNKS):
                out_w_v[r, pl.ds(v * LANES, LANES)] = zeros
        out_t_cp = pltpu.async_copy(
            out_t_v, out_t_hbm.at[pl.ds(base, ROWS_PER_WORKER)], sem_out)
        w_cp.wait()
        for r in range(ROWS_PER_WORKER):
            r_g = base + r
            r_eff16 = jnp.minimum(jnp.full((LANES,), r_g, jnp.int32),
                                  NUM_CLASSES - 1)
            w_row = plsc.load_gather(w_v, [jnp.full((LANES,), r, jnp.int32),
                                           iota4])
            mval = plsc.load_gather(mask_tab, [r_eff16])
            col = r_eff16 * BOX_DIM + iota4
            lane_mask = (iota < BOX_DIM) & (jnp.full((LANES,), r_g, jnp.int32)
                                            < NUM_CLASSES)
            plsc.store_scatter(out_w_v,
                               [jnp.full((LANES,), r, jnp.int32), col],
                               w_row * mval, mask=lane_mask)

        out_w_cp = pltpu.async_copy(
            out_w_v, out_w_hbm.at[pl.ds(base, ROWS_PER_WORKER)], sem_out)
        out_t_cp.wait()
        out_w_cp.wait()


@jax.jit
def kernel(bbox_targets, bbox_weights, labels):
    mesh = plsc.VectorSubcoreMesh(core_axis_name="c", subcore_axis_name="s")
    return pl.kernel(
        _body,
        out_type=(jax.ShapeDtypeStruct((M, OUT_W), jnp.float32),
                  jax.ShapeDtypeStruct((M, OUT_W), jnp.float32)),
        mesh=mesh,
        compiler_params=pltpu.CompilerParams(use_tc_tiling_on_sc=False,
                                             needs_layout_passes=False,
                                             skip_device_barrier=True),
        scratch_types=[
            pltpu.VMEM((8,), jnp.int32),
            pltpu.VMEM((NUM_CLASSES,), jnp.float32),
            pltpu.VMEM((ROWS_PER_WORKER, BOX_DIM), jnp.float32),
            pltpu.VMEM((ROWS_PER_WORKER, BOX_DIM), jnp.float32),
            pltpu.VMEM((ROWS_PER_WORKER, OUT_W), jnp.float32),
            pltpu.VMEM((ROWS_PER_WORKER, OUT_W), jnp.float32),
            pltpu.SemaphoreType.DMA,
            pltpu.SemaphoreType.DMA,
            pltpu.SemaphoreType.DMA,
            pltpu.SemaphoreType.DMA,
        ],
    )(bbox_targets, bbox_weights, labels)


# speedup vs baseline: 1.6488x; 1.0396x over previous
import functools

import jax
import jax.numpy as jnp
from jax import lax
from jax.experimental import pallas as pl
from jax.experimental.pallas import tpu as pltpu
from jax.experimental.pallas import tpu_sc as plsc

M = 300
NUM_CLASSES = 80
BOX_DIM = 4
OUT_W = NUM_CLASSES * BOX_DIM
ROWS_PER_WORKER = 10
NUM_WORKERS = 32
ACTIVE_WORKERS = M // ROWS_PER_WORKER
LANES = 16
CHUNKS = OUT_W // LANES


def _body(t_hbm, w_hbm, labels_hbm, out_t_hbm, out_w_hbm,
          labels_v, mask_tab, t_v, w_v, out_t_v, out_w_v,
          sem_lab, sem_t, sem_w, sem_out):
    nc = 2
    wid = lax.axis_index("s") * nc + lax.axis_index("c")
    base = wid * ROWS_PER_WORKER

    iota = lax.broadcasted_iota(jnp.int32, (LANES,), 0)
    iota4 = iota & 3
    zeros = jnp.zeros((LANES,), jnp.float32)
    ones = jnp.ones((LANES,), jnp.float32)

    @pl.when(wid < ACTIVE_WORKERS)
    def _():
        lab_cp = pltpu.async_copy(labels_hbm, labels_v, sem_lab)
        t_cp = pltpu.async_copy(t_hbm.at[pl.ds(base, ROWS_PER_WORKER)],
                                t_v, sem_t)
        w_cp = pltpu.async_copy(w_hbm.at[pl.ds(base, ROWS_PER_WORKER)],
                                w_v, sem_w)

        for i in range(NUM_CLASSES // LANES):
            mask_tab[pl.ds(i * LANES, LANES)] = zeros
        lab_cp.wait()
        lab_vec = plsc.load_gather(labels_v, [iota & 7])
        plsc.store_scatter(mask_tab, [lab_vec], ones)

        m = [plsc.load_gather(mask_tab, [(i * LANES + iota) >> 2])
             for i in range(CHUNKS)]

        t_cp.wait()
        w_cp.wait()

        def row_body(r, carry):
            r16 = jnp.full((LANES,), r, jnp.int32)
            t_row = plsc.load_gather(t_v, [r16, iota4])
            for v in range(CHUNKS):
                out_t_v[r, pl.ds(v * LANES, LANES)] = m[v] * t_row
            for v in range(CHUNKS):
                out_w_v[r, pl.ds(v * LANES, LANES)] = zeros
            rg16 = r16 + base
            r_eff16 = jnp.minimum(rg16, NUM_CLASSES - 1)
            w_row = plsc.load_gather(w_v, [r16, iota4])
            mval = plsc.load_gather(mask_tab, [r_eff16])
            col = r_eff16 * BOX_DIM + iota4
            lane_mask = (iota < BOX_DIM) & (rg16 < NUM_CLASSES)
            plsc.store_scatter(out_w_v, [r16, col], w_row * mval,
                               mask=lane_mask)
            return carry

        lax.fori_loop(0, ROWS_PER_WORKER, row_body, 0)

        out_t_cp = pltpu.async_copy(
            out_t_v, out_t_hbm.at[pl.ds(base, ROWS_PER_WORKER)], sem_out)
        out_w_cp = pltpu.async_copy(
            out_w_v, out_w_hbm.at[pl.ds(base, ROWS_PER_WORKER)], sem_out)
        out_t_cp.wait()
        out_w_cp.wait()


@jax.jit
def kernel(bbox_targets, bbox_weights, labels):
    mesh = plsc.VectorSubcoreMesh(core_axis_name="c", subcore_axis_name="s")
    return pl.kernel(
        _body,
        out_type=(jax.ShapeDtypeStruct((M, OUT_W), jnp.float32),
                  jax.ShapeDtypeStruct((M, OUT_W), jnp.float32)),
        mesh=mesh,
        compiler_params=pltpu.CompilerParams(use_tc_tiling_on_sc=False,
                                             needs_layout_passes=False,
                                             skip_device_barrier=True),
        scratch_types=[
            pltpu.VMEM((8,), jnp.int32),
            pltpu.VMEM((NUM_CLASSES,), jnp.float32),
            pltpu.VMEM((ROWS_PER_WORKER, BOX_DIM), jnp.float32),
            pltpu.VMEM((ROWS_PER_WORKER, BOX_DIM), jnp.float32),
            pltpu.VMEM((ROWS_PER_WORKER, OUT_W), jnp.float32),
            pltpu.VMEM((ROWS_PER_WORKER, OUT_W), jnp.float32),
            pltpu.SemaphoreType.DMA,
            pltpu.SemaphoreType.DMA,
            pltpu.SemaphoreType.DMA,
            pltpu.SemaphoreType.DMA,
        ],
    )(bbox_targets, bbox_weights, labels)


# speedup vs baseline: 1.7245x; 1.0460x over previous
import functools

import jax
import jax.numpy as jnp
from jax import lax
from jax.experimental import pallas as pl
from jax.experimental.pallas import tpu as pltpu
from jax.experimental.pallas import tpu_sc as plsc

M = 300
NUM_CLASSES = 80
BOX_DIM = 4
OUT_W = NUM_CLASSES * BOX_DIM
ROWS_PER_WORKER = 20
NUM_WORKERS = 16
ACTIVE_WORKERS = M // ROWS_PER_WORKER
LANES = 16
CHUNKS = OUT_W // LANES


def _body(t_hbm, w_hbm, labels_hbm, out_t_hbm, out_w_hbm,
          labels_v, mask_tab, t_v, w_v, out_t_v, out_w_v,
          sem_lab, sem_t, sem_w, sem_out):
    nc = 1
    wid = lax.axis_index("s") * nc + lax.axis_index("c")
    base = wid * ROWS_PER_WORKER

    iota = lax.broadcasted_iota(jnp.int32, (LANES,), 0)
    iota4 = iota & 3
    zeros = jnp.zeros((LANES,), jnp.float32)
    ones = jnp.ones((LANES,), jnp.float32)

    @pl.when(wid < ACTIVE_WORKERS)
    def _():
        lab_cp = pltpu.async_copy(labels_hbm, labels_v, sem_lab)
        t_cp = pltpu.async_copy(t_hbm.at[pl.ds(base, ROWS_PER_WORKER)],
                                t_v, sem_t)
        w_cp = pltpu.async_copy(w_hbm.at[pl.ds(base, ROWS_PER_WORKER)],
                                w_v, sem_w)

        for i in range(NUM_CLASSES // LANES):
            mask_tab[pl.ds(i * LANES, LANES)] = zeros
        lab_cp.wait()
        lab_vec = plsc.load_gather(labels_v, [iota & 7])
        plsc.store_scatter(mask_tab, [lab_vec], ones)

        m = [plsc.load_gather(mask_tab, [(i * LANES + iota) >> 2])
             for i in range(CHUNKS)]

        t_cp.wait()
        w_cp.wait()

        def row_body(r, carry):
            r16 = jnp.full((LANES,), r, jnp.int32)
            t_row = plsc.load_gather(t_v, [r16, iota4])
            for v in range(CHUNKS):
                out_t_v[r, pl.ds(v * LANES, LANES)] = m[v] * t_row
            for v in range(CHUNKS):
                out_w_v[r, pl.ds(v * LANES, LANES)] = zeros
            rg16 = r16 + base
            r_eff16 = jnp.minimum(rg16, NUM_CLASSES - 1)
            w_row = plsc.load_gather(w_v, [r16, iota4])
            mval = plsc.load_gather(mask_tab, [r_eff16])
            col = r_eff16 * BOX_DIM + iota4
            lane_mask = (iota < BOX_DIM) & (rg16 < NUM_CLASSES)
            plsc.store_scatter(out_w_v, [r16, col], w_row * mval,
                               mask=lane_mask)
            return carry

        lax.fori_loop(0, ROWS_PER_WORKER, row_body, 0)

        out_t_cp = pltpu.async_copy(
            out_t_v, out_t_hbm.at[pl.ds(base, ROWS_PER_WORKER)], sem_out)
        out_w_cp = pltpu.async_copy(
            out_w_v, out_w_hbm.at[pl.ds(base, ROWS_PER_WORKER)], sem_out)
        out_t_cp.wait()
        out_w_cp.wait()


@jax.jit
def kernel(bbox_targets, bbox_weights, labels):
    mesh = plsc.VectorSubcoreMesh(core_axis_name="c", subcore_axis_name="s",
                                  num_cores=1)
    return pl.kernel(
        _body,
        out_type=(jax.ShapeDtypeStruct((M, OUT_W), jnp.float32),
                  jax.ShapeDtypeStruct((M, OUT_W), jnp.float32)),
        mesh=mesh,
        compiler_params=pltpu.CompilerParams(use_tc_tiling_on_sc=False,
                                             needs_layout_passes=False,
                                             skip_device_barrier=True),
        scratch_types=[
            pltpu.VMEM((8,), jnp.int32),
            pltpu.VMEM((NUM_CLASSES,), jnp.float32),
            pltpu.VMEM((ROWS_PER_WORKER, BOX_DIM), jnp.float32),
            pltpu.VMEM((ROWS_PER_WORKER, BOX_DIM), jnp.float32),
            pltpu.VMEM((ROWS_PER_WORKER, OUT_W), jnp.float32),
            pltpu.VMEM((ROWS_PER_WORKER, OUT_W), jnp.float32),
            pltpu.SemaphoreType.DMA,
            pltpu.SemaphoreType.DMA,
            pltpu.SemaphoreType.DMA,
            pltpu.SemaphoreType.DMA,
        ],
    )(bbox_targets, bbox_weights, labels)
